# baseline (speedup 1.0000x reference)
import functools
import jax
import jax.numpy as jnp
from jax.experimental import pallas as pl
from jax.experimental.pallas import tpu as pltpu

N = 50000
E = 800000
G = 4096
IN_DIM = 39
EDGE_DIM = 8
H = 128
HEADS = 4
DH = 32
NUM_TASKS = 12


def _bn(x, g, b, eps=1e-5):
    m = jnp.mean(x, axis=0)
    v = jnp.var(x, axis=0)
    return (x - m) / jnp.sqrt(v + eps) * g + b


def _gatv2(x, src, dst, ea, Wl, bl, Wr, br, We, att, bc):
    xl = (x @ Wl + bl).reshape(-1, HEADS, DH)
    xr = (x @ Wr + br).reshape(-1, HEADS, DH)
    ee = (ea @ We).reshape(-1, HEADS, DH)
    m = xl[src] + xr[dst] + ee
    m = jax.nn.leaky_relu(m, negative_slope=0.2)
    alpha = jnp.sum(m * att[None, :, :], axis=-1)
    amax = jax.ops.segment_max(alpha, dst, num_segments=N)
    amax = jnp.where(jnp.isfinite(amax), amax, 0.0)
    ex = jnp.exp(alpha - amax[dst])
    den = jax.ops.segment_sum(ex, dst, num_segments=N)
    a = ex / (den[dst] + 1e-16)
    msg = xl[src] * a[:, :, None]
    out = jax.ops.segment_sum(msg, dst, num_segments=N)
    return out.reshape(N, HEADS * DH) + bc


def _mm_kernel(x_ref, w_ref, b_ref, o_ref):
    o_ref[...] = jnp.dot(x_ref[...], w_ref[...],
                         preferred_element_type=jnp.float32) + b_ref[...]


def _pallas_mm(x, w, b):
    m, k = x.shape
    n = w.shape[1]
    return pl.pallas_call(
        _mm_kernel,
        out_shape=jax.ShapeDtypeStruct((m, n), jnp.float32),
    )(x, w, b[None, :])


def kernel(x, edge_index, edge_attr, batch, p_in, conv1, conv2, conv3, p_trunk, p_heads):
    src = edge_index[0]
    dst = edge_index[1]
    W_in, b_in, g_in, be_in = p_in
    xc = jax.nn.relu(_bn(x @ W_in + b_in, g_in, be_in))
    for cp in (conv1, conv2, conv3):
        Wl, bl, Wr, br, We, att, bc, g, b = cp
        h = _gatv2(xc, src, dst, edge_attr, Wl, bl, Wr, br, We, att, bc)
        h = jax.nn.elu(_bn(h, g, b))
        xc = xc + h
    ones = jnp.ones((N,), xc.dtype)
    cnt = jax.ops.segment_sum(ones, batch, num_segments=G)
    ssum = jax.ops.segment_sum(xc, batch, num_segments=G)
    mean = ssum / jnp.maximum(cnt, 1.0)[:, None]
    mx = jax.ops.segment_max(xc, batch, num_segments=G)
    mx = jnp.where(jnp.isfinite(mx), mx, 0.0)
    xg = jnp.concatenate([mean, mx, ssum], axis=1)
    Wt1, bt1, gt1, bet1, Wt2, bt2, gt2, bet2 = p_trunk
    t = jax.nn.relu(_bn(_pallas_mm(xg, Wt1, bt1), gt1, bet1))
    t = jax.nn.relu(_bn(_pallas_mm(t, Wt2, bt2), gt2, bet2))
    Wh, bh = p_heads
    return jax.nn.sigmoid(_pallas_mm(t, Wh, bh))


# speedup vs baseline: 12.4431x; 12.4431x over previous
import functools
import jax
import jax.numpy as jnp
from jax import lax
from jax.experimental import pallas as pl
from jax.experimental.pallas import tpu as pltpu
from jax.experimental.pallas import tpu_sc as plsc

N = 50000
E = 800000
G = 4096
IN_DIM = 39
H = 128
HEADS = 4
DH = 32
NUM_TASKS = 12

NP = 50176
NB = 1024
EB = 3200
NW = 32
EPW = E // NW
NCH = 195
TAIL = EPW - NCH * 128
RPT = NP // 16
GPW = G // NW
CW = 16
NEG = -1.0e30

_f32 = jnp.float32


def _mm_stats_body(x_ref, w_ref, b_ref, y_ref, s_ref, *, nvalid, rb):
    i = pl.program_id(0)
    y = jnp.dot(x_ref[...], w_ref[...], preferred_element_type=_f32)
    y = y + b_ref[...]
    if nvalid is not None:
        row = lax.broadcasted_iota(jnp.int32, y.shape, 0) + i * rb
        y = jnp.where(row < nvalid, y, 0.0)
    y_ref[...] = y

    @pl.when(i == 0)
    def _():
        s_ref[...] = jnp.zeros_like(s_ref)

    su = jnp.sum(y, axis=0, keepdims=True)
    sq = jnp.sum(y * y, axis=0, keepdims=True)
    z = jnp.zeros((6, y.shape[1]), _f32)
    s_ref[...] += jnp.concatenate([su, sq, z], axis=0)


def _mm_stats(x, w, b, rb, nvalid):
    r, k = x.shape
    m = w.shape[1]
    grid = r // rb
    return pl.pallas_call(
        functools.partial(_mm_stats_body, nvalid=nvalid, rb=rb),
        grid=(grid,),
        in_specs=[
            pl.BlockSpec((rb, k), lambda i: (i, 0)),
            pl.BlockSpec((k, m), lambda i: (0, 0)),
            pl.BlockSpec((1, m), lambda i: (0, 0)),
        ],
        out_specs=[
            pl.BlockSpec((rb, m), lambda i: (i, 0)),
            pl.BlockSpec((8, m), lambda i: (0, 0)),
        ],
        out_shape=[
            jax.ShapeDtypeStruct((r, m), _f32),
            jax.ShapeDtypeStruct((8, m), _f32),
        ],
    )(x, w, b[None, :])


def _bn_act_body(y_ref, s_ref, g_ref, b_ref, *rest, act, nvalid, rb, count,
                 has_res):
    if has_res:
        res_ref, o_ref = rest
    else:
        (o_ref,) = rest
    i = pl.program_id(0)
    mean = s_ref[0:1, :] / count
    var = s_ref[1:2, :] / count - mean * mean
    inv = lax.rsqrt(var + 1e-5)
    z = (y_ref[...] - mean) * inv * g_ref[...] + b_ref[...]
    if act == "relu":
        z = jnp.maximum(z, 0.0)
    elif act == "elu":
        z = jnp.where(z > 0.0, z, jnp.exp(jnp.minimum(z, 0.0)) - 1.0)
    if has_res:
        z = z + res_ref[...]
    if nvalid is not None:
        row = lax.broadcasted_iota(jnp.int32, z.shape, 0) + i * rb
        z = jnp.where(row < nvalid, z, 0.0)
    o_ref[...] = z


def _bn_act(y, s, g, b, rb, count, act, nvalid, res=None):
    r, m = y.shape
    grid = r // rb
    specs = [
        pl.BlockSpec((rb, m), lambda i: (i, 0)),
        pl.BlockSpec((8, m), lambda i: (0, 0)),
        pl.BlockSpec((1, m), lambda i: (0, 0)),
        pl.BlockSpec((1, m), lambda i: (0, 0)),
    ]
    args = [y, s, g[None, :], b[None, :]]
    if res is not None:
        specs.append(pl.BlockSpec((rb, m), lambda i: (i, 0)))
        args.append(res)
    return pl.pallas_call(
        functools.partial(_bn_act_body, act=act, nvalid=nvalid, rb=rb,
                          count=count, has_res=res is not None),
        grid=(grid,),
        in_specs=specs,
        out_specs=pl.BlockSpec((rb, m), lambda i: (i, 0)),
        out_shape=jax.ShapeDtypeStruct((r, m), _f32),
    )(*args)


def _alpha_body(gxl_ref, gxr_ref, ea_ref, we_ref, att_ref, p_ref, sel_ref,
                a_ref, gm_ref):
    i = pl.program_id(0)
    ee = jnp.dot(ea_ref[...], we_ref[...], preferred_element_type=_f32)
    m = gxl_ref[...] + gxr_ref[...] + ee
    m = jnp.where(m > 0.0, m, 0.2 * m)
    w = m * att_ref[...]
    alpha = jnp.dot(w, p_ref[...], preferred_element_type=_f32)
    a_ref[...] = alpha

    @pl.when(i == 0)
    def _():
        gm_ref[...] = jnp.full((8, H), NEG, _f32)

    bm = jnp.max(alpha, axis=0, keepdims=True)
    bmp = jnp.dot(bm, sel_ref[...], preferred_element_type=_f32)
    row = lax.broadcasted_iota(jnp.int32, (8, H), 0)
    col = lax.broadcasted_iota(jnp.int32, (8, H), 1)
    upd = jnp.where((row == 0) & (col < HEADS),
                    jnp.broadcast_to(bmp, (8, H)), NEG)
    gm_ref[...] = jnp.maximum(gm_ref[...], upd)


def _alpha(gxl, gxr, ea, we, att_flat, p_mat, sel4):
    grid = E // EB
    return pl.pallas_call(
        _alpha_body,
        grid=(grid,),
        in_specs=[
            pl.BlockSpec((EB, H), lambda i: (i, 0)),
            pl.BlockSpec((EB, H), lambda i: (i, 0)),
            pl.BlockSpec((EB, 8), lambda i: (i, 0)),
            pl.BlockSpec((8, H), lambda i: (0, 0)),
            pl.BlockSpec((1, H), lambda i: (0, 0)),
            pl.BlockSpec((H, HEADS), lambda i: (0, 0)),
            pl.BlockSpec((HEADS, H), lambda i: (0, 0)),
        ],
        out_specs=[
            pl.BlockSpec((EB, HEADS), lambda i: (i, 0)),
            pl.BlockSpec((8, H), lambda i: (0, 0)),
        ],
        out_shape=[
            jax.ShapeDtypeStruct((E, HEADS), _f32),
            jax.ShapeDtypeStruct((8, H), _f32),
        ],
    )(gxl, gxr, ea, we, att_flat, p_mat, sel4)


def _msg_body(gxl_ref, a_ref, gm_ref, seld_ref, exp_ref, ezs_ref, *out_refs):
    a4 = jnp.dot(gm_ref[0:1, :], seld_ref[...],
                 preferred_element_type=_f32)
    ex = jnp.exp(a_ref[...] - a4)
    ex128 = jnp.dot(ex, exp_ref[...], preferred_element_type=_f32)
    full = gxl_ref[...] * ex128
    for c in range(8):
        out_refs[c][...] = full[:, c * CW:(c + 1) * CW]
    out_refs[8][...] = jnp.dot(ex, ezs_ref[...], preferred_element_type=_f32)


def _msg(gxl, alpha, gmax, seldown, expand, exzsel):
    grid = E // EB
    cspec = pl.BlockSpec((EB, CW), lambda i: (i, 0))
    cshape = jax.ShapeDtypeStruct((E, CW), _f32)
    return pl.pallas_call(
        _msg_body,
        grid=(grid,),
        in_specs=[
            pl.BlockSpec((EB, H), lambda i: (i, 0)),
            pl.BlockSpec((EB, HEADS), lambda i: (i, 0)),
            pl.BlockSpec((8, H), lambda i: (0, 0)),
            pl.BlockSpec((H, HEADS), lambda i: (0, 0)),
            pl.BlockSpec((HEADS, H), lambda i: (0, 0)),
            pl.BlockSpec((HEADS, CW), lambda i: (0, 0)),
        ],
        out_specs=[cspec] * 9,
        out_shape=[cshape] * 9,
    )(gxl, alpha, gmax, seldown, expand, exzsel)


def _combine_body(p_ref, bc_ref, dex_ref, h_ref, s_ref):
    i = pl.program_id(0)
    p = p_ref[...]
    num = jnp.concatenate(
        [p[0, c] + p[1, c] for c in range(8)], axis=1)
    denz = p[0, 8] + p[1, 8]
    den = jnp.dot(denz, dex_ref[...], preferred_element_type=_f32)
    h = num / (den + 1e-16) + bc_ref[...]
    row = lax.broadcasted_iota(jnp.int32, h.shape, 0) + i * NB
    h = jnp.where(row < N, h, 0.0)
    h_ref[...] = h

    @pl.when(i == 0)
    def _():
        s_ref[...] = jnp.zeros_like(s_ref)

    su = jnp.sum(h, axis=0, keepdims=True)
    sq = jnp.sum(h * h, axis=0, keepdims=True)
    s_ref[...] += jnp.concatenate([su, sq, jnp.zeros((6, H), _f32)], axis=0)


def _combine(part, bc, dexpand):
    grid = NP // NB
    return pl.pallas_call(
        _combine_body,
        grid=(grid,),
        in_specs=[
            pl.BlockSpec((2, 9, NB, CW), lambda i: (0, 0, i, 0)),
            pl.BlockSpec((1, H), lambda i: (0, 0)),
            pl.BlockSpec((CW, H), lambda i: (0, 0)),
        ],
        out_specs=[
            pl.BlockSpec((NB, H), lambda i: (i, 0)),
            pl.BlockSpec((8, H), lambda i: (0, 0)),
        ],
        out_shape=[
            jax.ShapeDtypeStruct((NP, H), _f32),
            jax.ShapeDtypeStruct((8, H), _f32),
        ],
    )(part, bc[None, :], dexpand)


def _pool_combine_body(gs_ref, gm_ref, cnt_ref, xg_ref):
    cnt = cnt_ref[...][:, 0:1]
    mean = gs_ref[...] / jnp.maximum(cnt, 1.0)
    mx = jnp.where(cnt > 0.0, gm_ref[...], 0.0)
    xg_ref[...] = jnp.concatenate([mean, mx, gs_ref[...]], axis=1)


def _pool_combine(gsum, gmax, gcnt):
    rb = 512
    grid = G // rb
    return pl.pallas_call(
        _pool_combine_body,
        grid=(grid,),
        in_specs=[
            pl.BlockSpec((rb, H), lambda i: (i, 0)),
            pl.BlockSpec((rb, H), lambda i: (i, 0)),
            pl.BlockSpec((rb, 16), lambda i: (i, 0)),
        ],
        out_specs=pl.BlockSpec((rb, 3 * H), lambda i: (i, 0)),
        out_shape=jax.ShapeDtypeStruct((G, 3 * H), _f32),
    )(gsum, gmax, gcnt)


def _head_body(x_ref, w_ref, b_ref, o_ref):
    z = jnp.dot(x_ref[...], w_ref[...], preferred_element_type=_f32)
    z = z + b_ref[...]
    o_ref[...] = 1.0 / (1.0 + jnp.exp(-z))


def _head(x, w, b):
    rb = 512
    grid = G // rb
    k = x.shape[1]
    m = w.shape[1]
    return pl.pallas_call(
        _head_body,
        grid=(grid,),
        in_specs=[
            pl.BlockSpec((rb, k), lambda i: (i, 0)),
            pl.BlockSpec((k, m), lambda i: (0, 0)),
            pl.BlockSpec((1, m), lambda i: (0, 0)),
        ],
        out_specs=pl.BlockSpec((rb, m), lambda i: (i, 0)),
        out_shape=jax.ShapeDtypeStruct((G, m), _f32),
    )(x, w, b[None, :])


_MESH = plsc.VectorSubcoreMesh(core_axis_name="c", subcore_axis_name="s")


def _gather_body(xl_hbm, xr_hbm, si_hbm, di_hbm, gxl_hbm, gxr_hbm,
                 sidx, didx, bufl, bufr, tbl, tbr, sem1, sem2):
    c = lax.axis_index("c")
    s = lax.axis_index("s")
    w = s * 2 + c
    e0 = w * EPW
    pltpu.sync_copy(si_hbm.at[pl.ds(e0, EPW)], sidx)
    pltpu.sync_copy(di_hbm.at[pl.ds(e0, EPW)], didx)

    def body(i, carry):
        off = i * 128
        cpl = pltpu.async_copy(xl_hbm.at[sidx.at[pl.ds(off, 128)]], bufl,
                               sem1)
        cpr = pltpu.async_copy(xr_hbm.at[didx.at[pl.ds(off, 128)]], bufr,
                               sem2)
        cpl.wait()
        cpr.wait()
        pltpu.sync_copy(bufl, gxl_hbm.at[pl.ds(e0 + off, 128)])
        pltpu.sync_copy(bufr, gxr_hbm.at[pl.ds(e0 + off, 128)])
        return carry

    lax.fori_loop(0, NCH, body, 0)
    off = NCH * 128
    cpl = pltpu.async_copy(xl_hbm.at[sidx.at[pl.ds(off, TAIL)]], tbl, sem1)
    cpr = pltpu.async_copy(xr_hbm.at[didx.at[pl.ds(off, TAIL)]], tbr, sem2)
    cpl.wait()
    cpr.wait()
    pltpu.sync_copy(tbl, gxl_hbm.at[pl.ds(e0 + off, TAIL)])
    pltpu.sync_copy(tbr, gxr_hbm.at[pl.ds(e0 + off, TAIL)])


def _sc_gather(xl, xr, si, di):
    return pl.kernel(
        _gather_body,
        mesh=_MESH,
        out_type=[
            jax.ShapeDtypeStruct((E, H), _f32),
            jax.ShapeDtypeStruct((E, H), _f32),
        ],
        scratch_types=[
            pltpu.VMEM((EPW,), jnp.int32),
            pltpu.VMEM((EPW,), jnp.int32),
            pltpu.VMEM((128, H), _f32),
            pltpu.VMEM((128, H), _f32),
            pltpu.VMEM((TAIL, H), _f32),
            pltpu.VMEM((TAIL, H), _f32),
            pltpu.SemaphoreType.DMA,
            pltpu.SemaphoreType.DMA,
        ],
    )(xl, xr, si, di)


def _scatter_body(*refs):
    chs = refs[:9]
    (dim_hbm, dit_hbm, zero_hbm, p0_hbm, p1_hbm,
     didx2, didxt2, rb, rbt, acc) = refs[9:]
    c = lax.axis_index("c")
    s = lax.axis_index("s")
    w = s * 2 + c
    e0 = w * EPW
    pltpu.sync_copy(dim_hbm.at[w], didx2)
    pltpu.sync_copy(dit_hbm.at[w], didxt2.at[0])

    for ch in range(9):
        src = chs[ch]
        pltpu.sync_copy(zero_hbm.at[pl.ds(s * RPT, RPT)],
                        acc.at[pl.ds(s * RPT, RPT)])
        plsc.subcore_barrier()

        def body(j, carry):
            pltpu.sync_copy(src.at[pl.ds(e0 + j * 128, 128)], rb)
            pltpu.sync_copy(rb, acc.at[didx2.at[j]], add=True)
            return carry
        lax.fori_loop(0, NCH, body, 0)
        pltpu.sync_copy(src.at[pl.ds(e0 + NCH * 128, TAIL)], rbt)
        pltpu.sync_copy(rbt, acc.at[didxt2.at[0]], add=True)
        plsc.subcore_barrier()

        @pl.when(c == 0)
        def _():
            pltpu.sync_copy(acc.at[pl.ds(s * RPT, RPT)],
                            p0_hbm.at[ch, pl.ds(s * RPT, RPT)])

        @pl.when(c == 1)
        def _():
            pltpu.sync_copy(acc.at[pl.ds(s * RPT, RPT)],
                            p1_hbm.at[ch, pl.ds(s * RPT, RPT)])
        plsc.subcore_barrier()


def _sc_scatter(msgs, di_main, di_tail, zeros_nd):
    return pl.kernel(
        _scatter_body,
        mesh=_MESH,
        out_type=[
            jax.ShapeDtypeStruct((9, NP, CW), _f32),
            jax.ShapeDtypeStruct((9, NP, CW), _f32),
        ],
        scratch_types=[
            pltpu.VMEM((NCH, 128), jnp.int32),
            pltpu.VMEM((8, TAIL), jnp.int32),
            pltpu.VMEM((128, CW), _f32),
            pltpu.VMEM((TAIL, CW), _f32),
            pltpu.VMEM_SHARED((NP, CW), _f32),
        ],
    )(*msgs, di_main, di_tail, zeros_nd)


def _pool_body(xc_hbm, b_hbm, gs_hbm, gm_hbm, gc_hbm,
               accs, accm, cnt16, bbuf, rowbuf, bwin):
    c = lax.axis_index("c")
    s = lax.axis_index("s")
    w = s * 2 + c
    g0 = w * GPW

    def initb(i, carry):
        accs[pl.ds(i * 16, 16)] = jnp.zeros((16,), _f32)
        accm[pl.ds(i * 16, 16)] = jnp.full((16,), NEG, _f32)
        return carry
    lax.fori_loop(0, GPW * H // 16, initb, 0)

    def initc(i, carry):
        cnt16[pl.ds(i * 16, 16)] = jnp.zeros((16,), _f32)
        return carry
    lax.fori_loop(0, GPW, initc, 0)

    def scan_body(ci, carry):
        stv, env = carry
        pltpu.sync_copy(b_hbm.at[pl.ds(ci * 1024, 1024)], bbuf)
        one = jnp.ones((16,), jnp.int32)
        zero = jnp.zeros((16,), jnp.int32)
        for k in range(64):
            v = bbuf[pl.ds(k * 16, 16)]
            stv = stv + jnp.where(v < g0, one, zero)
            env = env + jnp.where(v < g0 + GPW, one, zero)
        return stv, env

    z16 = jnp.zeros((16,), jnp.int32)
    stv, env = lax.fori_loop(0, NP // 1024, scan_body, (z16, z16))
    st = stv[0]
    en = env[0]
    for k in range(1, 16):
        st = st + stv[k]
        en = en + env[k]

    base0 = (st // 16) * 16
    nchunks = (en - base0 + 31) // 32
    one0 = jnp.where(lax.iota(jnp.int32, 16) == 0, 1.0, 0.0)

    def chunk_body(ci, carry):
        base = base0 + ci * 32
        pltpu.sync_copy(xc_hbm.at[pl.ds(base * H, 32 * H)], rowbuf)
        pltpu.sync_copy(b_hbm.at[pl.ds(base, 32)], bwin)

        for g16 in range(2):
            gv = bwin[pl.ds(g16 * 16, 16)]
            for lane in range(16):
                j = g16 * 16 + lane
                nid = base + j
                valid = (nid >= st) & (nid < en)
                g = gv[lane]

                @pl.when(valid)
                def _(g=g, j=j):
                    goff = (g - g0) * H
                    for k in range(8):
                        r = rowbuf[pl.ds(j * H + k * 16, 16)]
                        so = accs[pl.ds(goff + k * 16, 16)]
                        accs[pl.ds(goff + k * 16, 16)] = so + r
                        mo = accm[pl.ds(goff + k * 16, 16)]
                        accm[pl.ds(goff + k * 16, 16)] = jnp.maximum(mo, r)
                    co = (g - g0) * 16
                    cv = cnt16[pl.ds(co, 16)]
                    cnt16[pl.ds(co, 16)] = cv + one0
        return carry

    lax.fori_loop(0, nchunks, chunk_body, 0)

    pltpu.sync_copy(accs, gs_hbm.at[pl.ds(g0 * H, GPW * H)])
    pltpu.sync_copy(accm, gm_hbm.at[pl.ds(g0 * H, GPW * H)])
    pltpu.sync_copy(cnt16, gc_hbm.at[pl.ds(g0 * 16, GPW * 16)])


def _sc_pool(xc_flat, batch_pad):
    return pl.kernel(
        _pool_body,
        mesh=_MESH,
        out_type=[
            jax.ShapeDtypeStruct((G * H,), _f32),
            jax.ShapeDtypeStruct((G * H,), _f32),
            jax.ShapeDtypeStruct((G * 16,), _f32),
        ],
        scratch_types=[
            pltpu.VMEM((GPW * H,), _f32),
            pltpu.VMEM((GPW * H,), _f32),
            pltpu.VMEM((GPW * 16,), _f32),
            pltpu.VMEM((1024,), jnp.int32),
            pltpu.VMEM((32 * H,), _f32),
            pltpu.VMEM((32,), jnp.int32),
        ],
    )(xc_flat, batch_pad)


def kernel(x, edge_index, edge_attr, batch, p_in, conv1, conv2, conv3,
           p_trunk, p_heads):
    si = edge_index[0]
    di = edge_index[1]

    di_w = di.reshape(NW, EPW)
    di_main = di_w[:, :NCH * 128].reshape(NW, NCH, 128)
    di_tail = di_w[:, NCH * 128:]
    si_w = si.reshape(NW, EPW)
    si_main = si_w[:, :NCH * 128].reshape(NW, NCH, 128)
    si_tail = si_w[:, NCH * 128:]

    bpad = jnp.concatenate([batch, jnp.full((NP - N,), G, jnp.int32)])
    zeros_nd = jnp.zeros((NP, CW), _f32)

    p_mat = (jnp.arange(H)[:, None] // DH ==
             jnp.arange(HEADS)[None, :]).astype(_f32)
    expand = p_mat.T
    sel4 = jnp.eye(HEADS, H, dtype=_f32)
    seldown = jnp.eye(H, HEADS, dtype=_f32)
    exzsel = jnp.eye(HEADS, CW, dtype=_f32)
    dexpand = jnp.concatenate(
        [expand, jnp.zeros((CW - HEADS, H), _f32)], axis=0)

    W_in, b_in, g_in, be_in = p_in
    xp = jnp.zeros((NP, 48), _f32).at[:N, :IN_DIM].set(x)
    wp = jnp.zeros((48, H), _f32).at[:IN_DIM].set(W_in)
    y0, s0 = _mm_stats(xp, wp, b_in, NB, N)
    xc = _bn_act(y0, s0, g_in, be_in, NB, float(N), "relu", N)

    for cp in (conv1, conv2, conv3):
        Wl, bl, Wr, br, We, att, bc, gg, bb = cp
        xl, _ = _mm_stats(xc, Wl, bl, NB, N)
        xr, _ = _mm_stats(xc, Wr, br, NB, N)
        gxl, gxr = _sc_gather(xl, xr, si, di)
        wep = jnp.zeros((8, H), _f32).at[:, :].set(We)
        alpha, gmax = _alpha(gxl, gxr, edge_attr, wep,
                             att.reshape(1, H), p_mat, sel4)
        msgs = _msg(gxl, alpha, gmax, seldown, expand, exzsel)
        stk = jnp.concatenate(msgs, axis=1)
        seg = jax.ops.segment_sum(stk, di, num_segments=NP)
        part = jnp.zeros((2, 9, NP, CW), _f32).at[0].set(
            seg.reshape(NP, 9, CW).transpose(1, 0, 2))
        h, sh = _combine(part, bc, dexpand)
        xc = _bn_act(h, sh, gg, bb, NB, float(N), "elu", N, res=xc)

    xcn = xc[:N]
    gsum = jax.ops.segment_sum(xcn, batch, num_segments=G)
    gmax = jax.ops.segment_max(xcn, batch, num_segments=G)
    gmax = jnp.where(jnp.isfinite(gmax), gmax, 0.0)
    gcnt = jax.ops.segment_sum(jnp.ones((N,), _f32), batch, num_segments=G)
    gcnt16 = jnp.broadcast_to(gcnt[:, None], (G, 16))
    xg = _pool_combine(gsum, jnp.where(gcnt[:, None] > 0, gmax, 0.0), gcnt16)

    Wt1, bt1, gt1, bet1, Wt2, bt2, gt2, bet2 = p_trunk
    t1, st1 = _mm_stats(xg, Wt1, bt1, 512, None)
    t1n = _bn_act(t1, st1, gt1, bet1, 512, float(G), "relu", None)
    t2, st2 = _mm_stats(t1n, Wt2, bt2, 512, None)
    t2n = _bn_act(t2, st2, gt2, bet2, 512, float(G), "relu", None)

    Wh, bh = p_heads
    return _head(t2n, Wh, bh)
